# baseline (device time: 56754 ns/iter reference)
import jax
import jax.numpy as jnp
from jax import lax
from jax.experimental import pallas as pl
from jax.experimental.pallas import tpu as pltpu

N_DEV = 8
SQ = 512
SKV = 2048
D_MODEL = 1024
HQ_PER = 8
DH = 128
SCALE = 0.08838834764831843
ROWS_PER = SQ // N_DEV


def _fused(head0, x, Wq, K3, V3, Wo):

    def _copies(k_hbm, v_hbm, k_stage, v_stage, ksems, vsems, head, slot):
        kc = pltpu.make_async_copy(
            k_hbm.at[:, head, :], k_stage.at[slot], ksems.at[slot])
        vc = pltpu.make_async_copy(
            v_hbm.at[:, head, :], v_stage.at[slot], vsems.at[slot])
        return kc, vc

    def body(h0_ref, x_ref, wq_ref, k_hbm, v_hbm, wo_ref, out_ref,
             xb_ref, o_ref, stage_ref, rs_recv_ref,
             k_stage, v_stage, ksems, vsems,
             rs_send_sems, rs_recv_sems, ag_send_sems, ag_recv_sems):
        my = h0_ref[0] // HQ_PER
        h = pl.program_id(0)
        head = h0_ref[0] + h
        slot = lax.rem(h, 2)

        @pl.when(h == 0)
        def _():
            for c in _copies(k_hbm, v_hbm, k_stage, v_stage,
                             ksems, vsems, head, 0):
                c.start()
            for c in _copies(k_hbm, v_hbm, k_stage, v_stage,
                             ksems, vsems, head + 1, 1):
                c.start()
            xb_ref[...] = x_ref[...].astype(jnp.bfloat16)

        q = jnp.dot(xb_ref[...], wq_ref[...].astype(jnp.bfloat16),
                    preferred_element_type=jnp.float32) * SCALE
        q = q.astype(jnp.bfloat16)

        kwait, vwait = _copies(k_hbm, v_hbm, k_stage, v_stage,
                               ksems, vsems, head, slot)
        kwait.wait()
        k = k_stage[slot].astype(jnp.bfloat16)
        s = lax.dot_general(q, k, (((1,), (1,)), ((), ())),
                            preferred_element_type=jnp.float32)

        @pl.when(h < HQ_PER - 2)
        def _():
            kc, _vc = _copies(k_hbm, v_hbm, k_stage, v_stage,
                              ksems, vsems, head + 2, slot)
            kc.start()

        e = jnp.exp(s)
        l = jnp.sum(e, axis=-1, keepdims=True)

        vwait.wait()
        v = v_stage[slot].astype(jnp.bfloat16)
        o = jnp.dot(e.astype(jnp.bfloat16), v,
                    preferred_element_type=jnp.float32) / l

        @pl.when(h < HQ_PER - 2)
        def _():
            _kc, vc = _copies(k_hbm, v_hbm, k_stage, v_stage,
                              ksems, vsems, head + 2, slot)
            vc.start()

        o_ref[:, pl.ds(h * DH, DH)] = o.astype(jnp.bfloat16)

        @pl.when(h == HQ_PER - 1)
        def _():
            barrier_sem = pltpu.get_barrier_semaphore()
            for p in range(N_DEV):
                @pl.when(p != my)
                def _():
                    pl.semaphore_signal(
                        barrier_sem, inc=1,
                        device_id=(p,), device_id_type=pl.DeviceIdType.MESH,
                    )
            pl.semaphore_wait(barrier_sem, N_DEV - 1)

            out_ref[...] = jnp.dot(
                o_ref[...], wo_ref[...].astype(jnp.bfloat16),
                preferred_element_type=jnp.float32)
            stage_ref[...] = out_ref[...].astype(jnp.bfloat16)

            for p in range(N_DEV):
                @pl.when(p != my)
                def _():
                    rdma = pltpu.make_async_remote_copy(
                        src_ref=stage_ref.at[pl.ds(p * ROWS_PER, ROWS_PER), :],
                        dst_ref=rs_recv_ref.at[my],
                        send_sem=rs_send_sems.at[p],
                        recv_sem=rs_recv_sems.at[my],
                        device_id=(p,),
                        device_id_type=pl.DeviceIdType.MESH,
                    )
                    rdma.start()

            for s_id in range(N_DEV):
                @pl.when(s_id != my)
                def _():
                    recv = pltpu.make_async_remote_copy(
                        src_ref=stage_ref.at[pl.ds(0, ROWS_PER), :],
                        dst_ref=rs_recv_ref.at[s_id],
                        send_sem=rs_send_sems.at[s_id],
                        recv_sem=rs_recv_sems.at[s_id],
                        device_id=(s_id,),
                        device_id_type=pl.DeviceIdType.MESH,
                    )
                    recv.wait_recv()
                    out_ref[pl.ds(my * ROWS_PER, ROWS_PER), :] += (
                        rs_recv_ref[s_id].astype(jnp.float32))

            stage_ref[pl.ds(my * ROWS_PER, ROWS_PER), :] = (
                out_ref[pl.ds(my * ROWS_PER, ROWS_PER), :]
                .astype(jnp.bfloat16))
            for p in range(N_DEV):
                @pl.when(p != my)
                def _():
                    ag = pltpu.make_async_remote_copy(
                        src_ref=stage_ref.at[
                            pl.ds(my * ROWS_PER, ROWS_PER), :],
                        dst_ref=stage_ref.at[
                            pl.ds(my * ROWS_PER, ROWS_PER), :],
                        send_sem=ag_send_sems.at[p],
                        recv_sem=ag_recv_sems.at[my],
                        device_id=(p,),
                        device_id_type=pl.DeviceIdType.MESH,
                    )
                    ag.start()
            for s_id in range(N_DEV):
                @pl.when(s_id != my)
                def _():
                    agr = pltpu.make_async_remote_copy(
                        src_ref=stage_ref.at[pl.ds(0, ROWS_PER), :],
                        dst_ref=stage_ref.at[
                            pl.ds(s_id * ROWS_PER, ROWS_PER), :],
                        send_sem=ag_send_sems.at[my],
                        recv_sem=ag_recv_sems.at[s_id],
                        device_id=(s_id,),
                        device_id_type=pl.DeviceIdType.MESH,
                    )
                    agr.wait_recv()

            out_ref[...] = stage_ref[...].astype(jnp.float32)

            for p in range(N_DEV):
                @pl.when(p != my)
                def _():
                    for sems in (rs_send_sems, ag_send_sems):
                        d = pltpu.make_async_remote_copy(
                            src_ref=stage_ref.at[pl.ds(0, ROWS_PER), :],
                            dst_ref=rs_recv_ref.at[0],
                            send_sem=sems.at[p],
                            recv_sem=rs_recv_sems.at[0],
                            device_id=(0,),
                            device_id_type=pl.DeviceIdType.MESH,
                        )
                        d.wait_send()

    grid_spec = pltpu.PrefetchScalarGridSpec(
        num_scalar_prefetch=1,
        grid=(HQ_PER,),
        in_specs=[
            pl.BlockSpec((SQ, D_MODEL), lambda h, s: (0, 0)),
            pl.BlockSpec((D_MODEL, DH), lambda h, s: (0, h)),
            pl.BlockSpec(memory_space=pltpu.MemorySpace.HBM),
            pl.BlockSpec(memory_space=pltpu.MemorySpace.HBM),
            pl.BlockSpec((D_MODEL, D_MODEL), lambda h, s: (0, 0)),
        ],
        out_specs=pl.BlockSpec((SQ, D_MODEL), lambda h, s: (0, 0)),
        scratch_shapes=[
            pltpu.VMEM((SQ, D_MODEL), jnp.bfloat16),
            pltpu.VMEM((SQ, D_MODEL), jnp.bfloat16),
            pltpu.VMEM((SQ, D_MODEL), jnp.bfloat16),
            pltpu.VMEM((N_DEV, ROWS_PER, D_MODEL), jnp.bfloat16),
            pltpu.VMEM((2, SKV, DH), jnp.float32),
            pltpu.VMEM((2, SKV, DH), jnp.float32),
            pltpu.SemaphoreType.DMA((2,)),
            pltpu.SemaphoreType.DMA((2,)),
            pltpu.SemaphoreType.DMA((N_DEV,)),
            pltpu.SemaphoreType.DMA((N_DEV,)),
            pltpu.SemaphoreType.DMA((N_DEV,)),
            pltpu.SemaphoreType.DMA((N_DEV,)),
        ],
    )
    return pl.pallas_call(
        body,
        grid_spec=grid_spec,
        out_shape=jax.ShapeDtypeStruct((SQ, D_MODEL), jnp.float32),
        compiler_params=pltpu.CompilerParams(
            collective_id=0, dimension_semantics=("arbitrary",)),
    )(head0, x, Wq, K3, V3, Wo)


def kernel(x, Wq, Wo, K_ext, V_ext):
    my = lax.axis_index("i")
    head0 = jnp.reshape(my * HQ_PER, (1,)).astype(jnp.int32)
    out = _fused(head0, x[0], Wq, K_ext[0], V_ext[0], Wo)
    return out.reshape(1, SQ, D_MODEL)
